# baseline (speedup 1.0000x reference)
import functools

import jax
import jax.numpy as jnp
from jax import lax
from jax.experimental import pallas as pl
from jax.experimental.pallas import tpu as pltpu

_EPS = 1e-5


def _next_pow2(v):
    p = 1
    while p < v:
        p *= 2
    return p


def _lane_mask(c, nlanes, wq, w):
    ln = lax.broadcasted_iota(jnp.int32, (c, nlanes), 1)
    return (ln & (wq - 1)) < w


def _conv_cm(z, wt_ref, lo, h, wq, c):
    m = h * wq
    base = lo - wq - 1
    acc = None
    for dy in range(3):
        for dx in range(3):
            s = base + dy * wq + dx
            t = dy * 3 + dx
            part = jnp.dot(wt_ref[:, t * c:(t + 1) * c], z[:, s:s + m],
                           preferred_element_type=jnp.float32)
            acc = part if acc is None else acc + part
    return acc


def _bn_coeffs(s_ref, ss_ref, g_ref, b_ref, inv_m):
    tot = jnp.sum(s_ref[...], axis=0)
    tot2 = jnp.sum(ss_ref[...], axis=0)
    mean = tot * inv_m
    var = jnp.maximum(tot2 * inv_m - mean * mean, 0.0)
    inv = lax.rsqrt(var + _EPS)
    scale = g_ref[...] * inv
    shift = b_ref[...] - mean * scale
    return scale, shift


def _stage1(xp_ref, wt_ref, y_ref, s_ref, ss_ref, *, h, w, wq, c, g_imgs):
    m = h * wq
    mask = _lane_mask(c, m, wq, w)
    ssum = None
    ssq = None
    for g in range(g_imgs):
        acc = _conv_cm(xp_ref[g], wt_ref, wq + 1, h, wq, c)
        accm = jnp.where(mask, acc, 0.0)
        y_ref[g] = accm.astype(y_ref.dtype)
        ps = jnp.sum(accm, axis=1, keepdims=True)
        pss = jnp.sum(accm * accm, axis=1, keepdims=True)
        ssum = ps if ssum is None else ssum + ps
        ssq = pss if ssq is None else ssq + pss
    s_ref[0] = ssum
    ss_ref[0] = ssq


def _stage2(y1_ref, wt_ref, s_ref, ss_ref, g_ref, b_ref,
            y2_ref, s2_ref, ss2_ref, za_ref, zb_ref,
            *, h, w, wq, c, inv_m, g_imgs):
    m = h * wq
    zl = za_ref.shape[1]
    lo = wq + 1
    mask = _lane_mask(c, m, wq, w)
    scale, shift = _bn_coeffs(s_ref, ss_ref, g_ref, b_ref, inv_m)
    zero_lo = jnp.zeros((c, lo), za_ref.dtype)
    zero_hi = jnp.zeros((c, zl - lo - m), za_ref.dtype)
    ssum = None
    ssq = None
    for g in range(g_imgs):
        z_ref = za_ref if g % 2 == 0 else zb_ref
        zb = jnp.where(mask, jnp.maximum(y1_ref[g] * scale + shift, 0.0), 0.0)
        z_ref[:, 0:lo] = zero_lo
        z_ref[:, lo:lo + m] = zb.astype(z_ref.dtype)
        z_ref[:, lo + m:] = zero_hi
        acc = _conv_cm(z_ref[...], wt_ref, lo, h, wq, c)
        accm = jnp.where(mask, acc, 0.0)
        y2_ref[g] = accm.astype(y2_ref.dtype)
        ps = jnp.sum(accm, axis=1, keepdims=True)
        pss = jnp.sum(accm * accm, axis=1, keepdims=True)
        ssum = ps if ssum is None else ssum + ps
        ssq = pss if ssq is None else ssq + pss
    s2_ref[0] = ssum
    ss2_ref[0] = ssq


def _stage3(y2_ref, s_ref, ss_ref, g_ref, b_ref, xp_ref, o_ref,
            *, wq, m, inv_m, g_imgs):
    lo = wq + 1
    scale, shift = _bn_coeffs(s_ref, ss_ref, g_ref, b_ref, inv_m)
    for g in range(g_imgs):
        res = xp_ref[g][:, lo:lo + m]
        y = y2_ref[g] * scale + shift + res
        o_ref[g] = jnp.maximum(y, 0.0).astype(o_ref.dtype)


def _pick_div(n, cap):
    for d in range(cap, 0, -1):
        if n % d == 0:
            return d
    return 1


def kernel(x, w1, cb1, g1, b1, w2, cb2, g2, b2):
    del cb1, cb2
    n, c, h, w = x.shape
    wq = _next_pow2(w + 2)
    m = h * wq
    hw = h * w
    zl = -(-(m + 2 * wq + 2) // 128) * 128
    gi = _pick_div(n, 2)
    steps = n // gi

    xp = jnp.pad(x, ((0, 0), (0, 0), (1, 1), (1, wq - w - 1)))
    xp = xp.reshape(n, c, (h + 2) * wq)
    xp = jnp.pad(xp, ((0, 0), (0, 0), (0, zl - (h + 2) * wq)))
    xp = xp.astype(jnp.bfloat16)
    wt1 = w1.reshape(9 * c, c).T.astype(jnp.bfloat16)
    wt2 = w2.reshape(9 * c, c).T.astype(jnp.bfloat16)
    g1v = g1.astype(jnp.float32).reshape(c, 1)
    b1v = b1.astype(jnp.float32).reshape(c, 1)
    g2v = g2.astype(jnp.float32).reshape(c, 1)
    b2v = b2.astype(jnp.float32).reshape(c, 1)
    inv_m = 1.0 / (n * hw)

    zp_spec = pl.BlockSpec((gi, c, zl), lambda i: (i, 0, 0))
    pad_spec = pl.BlockSpec((gi, c, m), lambda i: (i, 0, 0))
    cmp_spec = pl.BlockSpec((gi, c, hw), lambda i: (i, 0, 0))
    stat_spec = pl.BlockSpec((1, c, 1), lambda i: (i, 0, 0))
    stat_full = pl.BlockSpec((steps, c, 1), lambda i: (0, 0, 0))
    w_spec = pl.BlockSpec((c, 9 * c), lambda i: (0, 0))
    vec_spec = pl.BlockSpec((c, 1), lambda i: (0, 0))
    params = pltpu.CompilerParams(dimension_semantics=("arbitrary",))
    f32 = jnp.float32
    bf16 = jnp.bfloat16

    y1, s1, ss1 = pl.pallas_call(
        functools.partial(_stage1, h=h, w=w, wq=wq, c=c, g_imgs=gi),
        grid=(steps,),
        in_specs=[zp_spec, w_spec],
        out_specs=(pad_spec, stat_spec, stat_spec),
        out_shape=(jax.ShapeDtypeStruct((n, c, m), bf16),
                   jax.ShapeDtypeStruct((steps, c, 1), f32),
                   jax.ShapeDtypeStruct((steps, c, 1), f32)),
        compiler_params=params,
    )(xp, wt1)

    y2, s2, ss2 = pl.pallas_call(
        functools.partial(_stage2, h=h, w=w, wq=wq, c=c, inv_m=inv_m,
                          g_imgs=gi),
        grid=(steps,),
        in_specs=[pad_spec, w_spec, stat_full, stat_full, vec_spec, vec_spec],
        out_specs=(pad_spec, stat_spec, stat_spec),
        out_shape=(jax.ShapeDtypeStruct((n, c, m), bf16),
                   jax.ShapeDtypeStruct((steps, c, 1), f32),
                   jax.ShapeDtypeStruct((steps, c, 1), f32)),
        scratch_shapes=[pltpu.VMEM((c, zl), jnp.bfloat16),
                        pltpu.VMEM((c, zl), jnp.bfloat16)],
        compiler_params=params,
    )(y1, wt2, s1, ss1, g1v, b1v)

    out = pl.pallas_call(
        functools.partial(_stage3, wq=wq, m=m, inv_m=inv_m, g_imgs=gi),
        grid=(steps,),
        in_specs=[pad_spec, stat_full, stat_full, vec_spec, vec_spec,
                  zp_spec],
        out_specs=pad_spec,
        out_shape=jax.ShapeDtypeStruct((n, c, m), f32),
        compiler_params=params,
    )(y2, s2, ss2, g2v, b2v, xp)

    return out.reshape(n, c, h, wq)[:, :, :, :w]


# speedup vs baseline: 1.1285x; 1.1285x over previous
import functools

import jax
import jax.numpy as jnp
from jax import lax
from jax.experimental import pallas as pl
from jax.experimental.pallas import tpu as pltpu

_EPS = 1e-5


def _next_pow2(v):
    p = 1
    while p < v:
        p *= 2
    return p


def _lane_mask(c, nlanes, wq, w):
    ln = lax.broadcasted_iota(jnp.int32, (c, nlanes), 1)
    return (ln & (wq - 1)) < w


def _conv_cm(z, wt_ref, lo, h, wq, c):
    m = h * wq
    base = lo - wq - 1
    acc = None
    for dy in range(3):
        for dx in range(3):
            s = base + dy * wq + dx
            t = dy * 3 + dx
            part = jnp.dot(wt_ref[:, t * c:(t + 1) * c], z[:, s:s + m],
                           preferred_element_type=jnp.float32)
            acc = part if acc is None else acc + part
    return acc


def _bn_coeffs(s_ref, ss_ref, g_ref, b_ref, inv_m):
    tot = jnp.sum(s_ref[...], axis=0)
    tot2 = jnp.sum(ss_ref[...], axis=0)
    mean = tot * inv_m
    var = jnp.maximum(tot2 * inv_m - mean * mean, 0.0)
    inv = lax.rsqrt(var + _EPS)
    scale = g_ref[...] * inv
    shift = b_ref[...] - mean * scale
    return scale, shift


def _stage1(xp_ref, wt_ref, y_ref, s_ref, ss_ref, *, h, w, wq, c, g_imgs):
    m = h * wq
    mask = _lane_mask(c, m, wq, w)
    ssum = None
    ssq = None
    for g in range(g_imgs):
        acc = _conv_cm(xp_ref[g], wt_ref, wq + 1, h, wq, c)
        accm = jnp.where(mask, acc, 0.0)
        y_ref[g] = accm.astype(y_ref.dtype)
        ps = jnp.sum(accm, axis=1, keepdims=True)
        pss = jnp.sum(accm * accm, axis=1, keepdims=True)
        ssum = ps if ssum is None else ssum + ps
        ssq = pss if ssq is None else ssq + pss
    s_ref[0] = ssum
    ss_ref[0] = ssq


def _stage2(y1_ref, wt_ref, s_ref, ss_ref, g_ref, b_ref,
            y2_ref, s2_ref, ss2_ref, za_ref, zb_ref,
            *, h, w, wq, c, inv_m, g_imgs):
    m = h * wq
    zl = za_ref.shape[1]
    lo = wq + 1
    mask = _lane_mask(c, m, wq, w)
    scale, shift = _bn_coeffs(s_ref, ss_ref, g_ref, b_ref, inv_m)
    zero_lo = jnp.zeros((c, lo), za_ref.dtype)
    zero_hi = jnp.zeros((c, zl - lo - m), za_ref.dtype)
    ssum = None
    ssq = None
    for g in range(g_imgs):
        z_ref = za_ref if g % 2 == 0 else zb_ref
        zb = jnp.where(mask, jnp.maximum(y1_ref[g] * scale + shift, 0.0), 0.0)
        z_ref[:, 0:lo] = zero_lo
        z_ref[:, lo:lo + m] = zb.astype(z_ref.dtype)
        z_ref[:, lo + m:] = zero_hi
        acc = _conv_cm(z_ref[...], wt_ref, lo, h, wq, c)
        accm = jnp.where(mask, acc, 0.0)
        y2_ref[g] = accm.astype(y2_ref.dtype)
        ps = jnp.sum(accm, axis=1, keepdims=True)
        pss = jnp.sum(accm * accm, axis=1, keepdims=True)
        ssum = ps if ssum is None else ssum + ps
        ssq = pss if ssq is None else ssq + pss
    s2_ref[0] = ssum
    ss2_ref[0] = ssq


def _stage3(y2_ref, s_ref, ss_ref, g_ref, b_ref, xp_ref, o_ref,
            *, wq, m, inv_m, g_imgs):
    lo = wq + 1
    scale, shift = _bn_coeffs(s_ref, ss_ref, g_ref, b_ref, inv_m)
    for g in range(g_imgs):
        res = xp_ref[g][:, lo:lo + m]
        y = y2_ref[g] * scale + shift + res
        o_ref[g] = jnp.maximum(y, 0.0).astype(o_ref.dtype)


def _pick_div(n, cap):
    for d in range(cap, 0, -1):
        if n % d == 0:
            return d
    return 1


def kernel(x, w1, cb1, g1, b1, w2, cb2, g2, b2):
    del cb1, cb2
    n, c, h, w = x.shape
    wq = _next_pow2(w + 2)
    m = h * wq
    hw = h * w
    zl = -(-(m + 2 * wq + 2) // 128) * 128
    gi = _pick_div(n, 4)
    steps = n // gi

    xp = jnp.pad(x, ((0, 0), (0, 0), (1, 1), (1, wq - w - 1)))
    xp = xp.reshape(n, c, (h + 2) * wq)
    xp = jnp.pad(xp, ((0, 0), (0, 0), (0, zl - (h + 2) * wq)))
    xp = xp.astype(jnp.bfloat16)
    wt1 = w1.reshape(9 * c, c).T.astype(jnp.bfloat16)
    wt2 = w2.reshape(9 * c, c).T.astype(jnp.bfloat16)
    g1v = g1.astype(jnp.float32).reshape(c, 1)
    b1v = b1.astype(jnp.float32).reshape(c, 1)
    g2v = g2.astype(jnp.float32).reshape(c, 1)
    b2v = b2.astype(jnp.float32).reshape(c, 1)
    inv_m = 1.0 / (n * hw)

    zp_spec = pl.BlockSpec((gi, c, zl), lambda i: (i, 0, 0))
    pad_spec = pl.BlockSpec((gi, c, m), lambda i: (i, 0, 0))
    cmp_spec = pl.BlockSpec((gi, c, hw), lambda i: (i, 0, 0))
    stat_spec = pl.BlockSpec((1, c, 1), lambda i: (i, 0, 0))
    stat_full = pl.BlockSpec((steps, c, 1), lambda i: (0, 0, 0))
    w_spec = pl.BlockSpec((c, 9 * c), lambda i: (0, 0))
    vec_spec = pl.BlockSpec((c, 1), lambda i: (0, 0))
    params = pltpu.CompilerParams(dimension_semantics=("arbitrary",))
    f32 = jnp.float32
    bf16 = jnp.bfloat16

    y1, s1, ss1 = pl.pallas_call(
        functools.partial(_stage1, h=h, w=w, wq=wq, c=c, g_imgs=gi),
        grid=(steps,),
        in_specs=[zp_spec, w_spec],
        out_specs=(pad_spec, stat_spec, stat_spec),
        out_shape=(jax.ShapeDtypeStruct((n, c, m), bf16),
                   jax.ShapeDtypeStruct((steps, c, 1), f32),
                   jax.ShapeDtypeStruct((steps, c, 1), f32)),
        compiler_params=params,
    )(xp, wt1)

    y2, s2, ss2 = pl.pallas_call(
        functools.partial(_stage2, h=h, w=w, wq=wq, c=c, inv_m=inv_m,
                          g_imgs=gi),
        grid=(steps,),
        in_specs=[pad_spec, w_spec, stat_full, stat_full, vec_spec, vec_spec],
        out_specs=(pad_spec, stat_spec, stat_spec),
        out_shape=(jax.ShapeDtypeStruct((n, c, m), bf16),
                   jax.ShapeDtypeStruct((steps, c, 1), f32),
                   jax.ShapeDtypeStruct((steps, c, 1), f32)),
        scratch_shapes=[pltpu.VMEM((c, zl), jnp.bfloat16),
                        pltpu.VMEM((c, zl), jnp.bfloat16)],
        compiler_params=params,
    )(y1, wt2, s1, ss1, g1v, b1v)

    out = pl.pallas_call(
        functools.partial(_stage3, wq=wq, m=m, inv_m=inv_m, g_imgs=gi),
        grid=(steps,),
        in_specs=[pad_spec, stat_full, stat_full, vec_spec, vec_spec,
                  zp_spec],
        out_specs=pad_spec,
        out_shape=jax.ShapeDtypeStruct((n, c, m), f32),
        compiler_params=params,
    )(y2, s2, ss2, g2v, b2v, xp)

    return out.reshape(n, c, h, wq)[:, :, :, :w]


# speedup vs baseline: 1.1524x; 1.0212x over previous
import functools

import jax
import jax.numpy as jnp
from jax import lax
from jax.experimental import pallas as pl
from jax.experimental.pallas import tpu as pltpu

_EPS = 1e-5


def _next_pow2(v):
    p = 1
    while p < v:
        p *= 2
    return p


def _lane_mask(c, nlanes, wq, w):
    ln = lax.broadcasted_iota(jnp.int32, (c, nlanes), 1)
    return (ln & (wq - 1)) < w


def _conv_cm(z, wt_ref, lo, h, wq, c):
    m = h * wq
    base = lo - wq - 1
    acc = None
    for dy in range(3):
        for dx in range(3):
            s = base + dy * wq + dx
            t = dy * 3 + dx
            part = jnp.dot(wt_ref[:, t * c:(t + 1) * c], z[:, s:s + m],
                           preferred_element_type=jnp.float32)
            acc = part if acc is None else acc + part
    return acc


def _sum_wide(v, nlanes):
    if nlanes % 128 != 0:
        return jnp.sum(v, axis=1, keepdims=True)
    acc = v[:, 0:128]
    for i in range(1, nlanes // 128):
        acc = acc + v[:, i * 128:(i + 1) * 128]
    return acc


def _bn_coeffs(s_ref, ss_ref, g_ref, b_ref, inv_m):
    tot = jnp.sum(s_ref[...], axis=0)
    tot2 = jnp.sum(ss_ref[...], axis=0)
    mean = tot * inv_m
    var = jnp.maximum(tot2 * inv_m - mean * mean, 0.0)
    inv = lax.rsqrt(var + _EPS)
    scale = g_ref[...] * inv
    shift = b_ref[...] - mean * scale
    return scale, shift


def _stage1(xp_ref, wt_ref, y_ref, s_ref, ss_ref, *, h, w, wq, c, g_imgs):
    m = h * wq
    mask = _lane_mask(c, m, wq, w)
    ssum = None
    ssq = None
    for g in range(g_imgs):
        acc = _conv_cm(xp_ref[g], wt_ref, wq + 1, h, wq, c)
        accm = jnp.where(mask, acc, 0.0)
        y_ref[g] = accm.astype(y_ref.dtype)
        ps = _sum_wide(accm, m)
        pss = _sum_wide(accm * accm, m)
        ssum = ps if ssum is None else ssum + ps
        ssq = pss if ssq is None else ssq + pss
    s_ref[0] = jnp.sum(ssum, axis=1, keepdims=True)
    ss_ref[0] = jnp.sum(ssq, axis=1, keepdims=True)


def _stage2(y1_ref, wt_ref, s_ref, ss_ref, g_ref, b_ref,
            y2_ref, s2_ref, ss2_ref, za_ref, zb_ref,
            *, h, w, wq, c, inv_m, g_imgs):
    m = h * wq
    zl = za_ref.shape[1]
    lo = wq + 1
    mask = _lane_mask(c, m, wq, w)
    scale, shift = _bn_coeffs(s_ref, ss_ref, g_ref, b_ref, inv_m)
    zero_lo = jnp.zeros((c, lo), za_ref.dtype)
    zero_hi = jnp.zeros((c, zl - lo - m), za_ref.dtype)
    ssum = None
    ssq = None
    for g in range(g_imgs):
        z_ref = za_ref if g % 2 == 0 else zb_ref
        zb = jnp.where(mask, jnp.maximum(y1_ref[g] * scale + shift, 0.0), 0.0)
        z_ref[:, 0:lo] = zero_lo
        z_ref[:, lo:lo + m] = zb.astype(z_ref.dtype)
        z_ref[:, lo + m:] = zero_hi
        acc = _conv_cm(z_ref[...], wt_ref, lo, h, wq, c)
        accm = jnp.where(mask, acc, 0.0)
        y2_ref[g] = accm.astype(y2_ref.dtype)
        ps = _sum_wide(accm, m)
        pss = _sum_wide(accm * accm, m)
        ssum = ps if ssum is None else ssum + ps
        ssq = pss if ssq is None else ssq + pss
    s2_ref[0] = jnp.sum(ssum, axis=1, keepdims=True)
    ss2_ref[0] = jnp.sum(ssq, axis=1, keepdims=True)


def _stage3(y2_ref, s_ref, ss_ref, g_ref, b_ref, xp_ref, o_ref,
            *, wq, m, inv_m, g_imgs):
    lo = wq + 1
    scale, shift = _bn_coeffs(s_ref, ss_ref, g_ref, b_ref, inv_m)
    for g in range(g_imgs):
        res = xp_ref[g][:, lo:lo + m]
        y = y2_ref[g] * scale + shift + res
        o_ref[g] = jnp.maximum(y, 0.0).astype(o_ref.dtype)


def _pick_div(n, cap):
    for d in range(cap, 0, -1):
        if n % d == 0:
            return d
    return 1


def kernel(x, w1, cb1, g1, b1, w2, cb2, g2, b2):
    del cb1, cb2
    n, c, h, w = x.shape
    wq = _next_pow2(w + 2)
    m = h * wq
    hw = h * w
    zl = -(-(m + 2 * wq + 2) // 128) * 128
    gi = _pick_div(n, 4)
    steps = n // gi

    xp = jnp.pad(x, ((0, 0), (0, 0), (1, 1), (1, wq - w - 1)))
    xp = xp.reshape(n, c, (h + 2) * wq)
    xp = jnp.pad(xp, ((0, 0), (0, 0), (0, zl - (h + 2) * wq)))
    xp = xp.astype(jnp.bfloat16)
    wt1 = w1.reshape(9 * c, c).T.astype(jnp.bfloat16)
    wt2 = w2.reshape(9 * c, c).T.astype(jnp.bfloat16)
    g1v = g1.astype(jnp.float32).reshape(c, 1)
    b1v = b1.astype(jnp.float32).reshape(c, 1)
    g2v = g2.astype(jnp.float32).reshape(c, 1)
    b2v = b2.astype(jnp.float32).reshape(c, 1)
    inv_m = 1.0 / (n * hw)

    zp_spec = pl.BlockSpec((gi, c, zl), lambda i: (i, 0, 0))
    pad_spec = pl.BlockSpec((gi, c, m), lambda i: (i, 0, 0))
    cmp_spec = pl.BlockSpec((gi, c, hw), lambda i: (i, 0, 0))
    stat_spec = pl.BlockSpec((1, c, 1), lambda i: (i, 0, 0))
    stat_full = pl.BlockSpec((steps, c, 1), lambda i: (0, 0, 0))
    w_spec = pl.BlockSpec((c, 9 * c), lambda i: (0, 0))
    vec_spec = pl.BlockSpec((c, 1), lambda i: (0, 0))
    params = pltpu.CompilerParams(dimension_semantics=("arbitrary",))
    f32 = jnp.float32
    bf16 = jnp.bfloat16

    y1, s1, ss1 = pl.pallas_call(
        functools.partial(_stage1, h=h, w=w, wq=wq, c=c, g_imgs=gi),
        grid=(steps,),
        in_specs=[zp_spec, w_spec],
        out_specs=(pad_spec, stat_spec, stat_spec),
        out_shape=(jax.ShapeDtypeStruct((n, c, m), bf16),
                   jax.ShapeDtypeStruct((steps, c, 1), f32),
                   jax.ShapeDtypeStruct((steps, c, 1), f32)),
        compiler_params=params,
    )(xp, wt1)

    y2, s2, ss2 = pl.pallas_call(
        functools.partial(_stage2, h=h, w=w, wq=wq, c=c, inv_m=inv_m,
                          g_imgs=gi),
        grid=(steps,),
        in_specs=[pad_spec, w_spec, stat_full, stat_full, vec_spec, vec_spec],
        out_specs=(pad_spec, stat_spec, stat_spec),
        out_shape=(jax.ShapeDtypeStruct((n, c, m), bf16),
                   jax.ShapeDtypeStruct((steps, c, 1), f32),
                   jax.ShapeDtypeStruct((steps, c, 1), f32)),
        scratch_shapes=[pltpu.VMEM((c, zl), jnp.bfloat16),
                        pltpu.VMEM((c, zl), jnp.bfloat16)],
        compiler_params=params,
    )(y1, wt2, s1, ss1, g1v, b1v)

    out = pl.pallas_call(
        functools.partial(_stage3, wq=wq, m=m, inv_m=inv_m, g_imgs=gi),
        grid=(steps,),
        in_specs=[pad_spec, stat_full, stat_full, vec_spec, vec_spec,
                  zp_spec],
        out_specs=pad_spec,
        out_shape=jax.ShapeDtypeStruct((n, c, m), f32),
        compiler_params=params,
    )(y2, s2, ss2, g2v, b2v, xp)

    return out.reshape(n, c, h, wq)[:, :, :, :w]


# speedup vs baseline: 1.1552x; 1.0024x over previous
import functools

import jax
import jax.numpy as jnp
from jax import lax
from jax.experimental import pallas as pl
from jax.experimental.pallas import tpu as pltpu

_EPS = 1e-5


def _next_pow2(v):
    p = 1
    while p < v:
        p *= 2
    return p


def _lane_mask(c, nlanes, wq, w):
    ln = lax.broadcasted_iota(jnp.int32, (c, nlanes), 1)
    return (ln & (wq - 1)) < w


def _conv_cm(z, wt_ref, lo, h, wq, c):
    m = h * wq
    base = lo - wq - 1
    acc = None
    for dy in range(3):
        for dx in range(3):
            s = base + dy * wq + dx
            t = dy * 3 + dx
            part = jnp.dot(wt_ref[:, t * c:(t + 1) * c], z[:, s:s + m],
                           preferred_element_type=jnp.float32)
            acc = part if acc is None else acc + part
    return acc


def _sum_wide(v, nlanes):
    if nlanes % 128 != 0:
        return jnp.sum(v, axis=1, keepdims=True)
    acc = v[:, 0:128]
    for i in range(1, nlanes // 128):
        acc = acc + v[:, i * 128:(i + 1) * 128]
    return acc


def _bn_coeffs(s_ref, ss_ref, g_ref, b_ref, inv_m):
    tot = jnp.sum(s_ref[...], axis=0)
    tot2 = jnp.sum(ss_ref[...], axis=0)
    mean = tot * inv_m
    var = jnp.maximum(tot2 * inv_m - mean * mean, 0.0)
    inv = lax.rsqrt(var + _EPS)
    scale = g_ref[...] * inv
    shift = b_ref[...] - mean * scale
    return scale, shift


def _stage1(xp_ref, wt_ref, y_ref, s_ref, ss_ref, *, h, w, wq, c, g_imgs):
    m = h * wq
    mask = _lane_mask(c, m, wq, w)
    ssum = None
    ssq = None
    for g in range(g_imgs):
        acc = _conv_cm(xp_ref[g], wt_ref, wq + 1, h, wq, c)
        accm = jnp.where(mask, acc, 0.0)
        y_ref[g] = accm.astype(y_ref.dtype)
        ps = _sum_wide(accm, m)
        pss = _sum_wide(accm * accm, m)
        ssum = ps if ssum is None else ssum + ps
        ssq = pss if ssq is None else ssq + pss
    s_ref[0] = jnp.sum(ssum, axis=1, keepdims=True)
    ss_ref[0] = jnp.sum(ssq, axis=1, keepdims=True)


def _stage2(y1_ref, wt_ref, s_ref, ss_ref, g_ref, b_ref,
            y2_ref, s2_ref, ss2_ref, za_ref, zb_ref,
            *, h, w, wq, c, inv_m, g_imgs):
    m = h * wq
    zl = za_ref.shape[1]
    lo = wq + 1
    mask = _lane_mask(c, m, wq, w)
    scale, shift = _bn_coeffs(s_ref, ss_ref, g_ref, b_ref, inv_m)
    zero_lo = jnp.zeros((c, lo), za_ref.dtype)
    zero_hi = jnp.zeros((c, zl - lo - m), za_ref.dtype)
    ssum = None
    ssq = None
    for g in range(g_imgs):
        z_ref = za_ref if g % 2 == 0 else zb_ref
        zb = jnp.where(mask, jnp.maximum(y1_ref[g] * scale + shift, 0.0), 0.0)
        z_ref[:, 0:lo] = zero_lo
        z_ref[:, lo:lo + m] = zb.astype(z_ref.dtype)
        z_ref[:, lo + m:] = zero_hi
        acc = _conv_cm(z_ref[...], wt_ref, lo, h, wq, c)
        accm = jnp.where(mask, acc, 0.0)
        y2_ref[g] = accm.astype(y2_ref.dtype)
        ps = _sum_wide(accm, m)
        pss = _sum_wide(accm * accm, m)
        ssum = ps if ssum is None else ssum + ps
        ssq = pss if ssq is None else ssq + pss
    s2_ref[0] = jnp.sum(ssum, axis=1, keepdims=True)
    ss2_ref[0] = jnp.sum(ssq, axis=1, keepdims=True)


def _stage3(y2_ref, s_ref, ss_ref, g_ref, b_ref, xp_ref, o_ref,
            *, wq, m, inv_m, g_imgs):
    lo = wq + 1
    scale, shift = _bn_coeffs(s_ref, ss_ref, g_ref, b_ref, inv_m)
    for g in range(g_imgs):
        res = xp_ref[g][:, lo:lo + m]
        y = y2_ref[g] * scale + shift + res
        o_ref[g] = jnp.maximum(y, 0.0).astype(o_ref.dtype)


def _pick_div(n, cap):
    for d in range(cap, 0, -1):
        if n % d == 0:
            return d
    return 1


def kernel(x, w1, cb1, g1, b1, w2, cb2, g2, b2):
    del cb1, cb2
    n, c, h, w = x.shape
    wq = _next_pow2(w + 2)
    m = h * wq
    hw = h * w
    zl = -(-(m + 2 * wq + 2) // 128) * 128
    gi = _pick_div(n, 4)
    steps = n // gi

    xp = jnp.pad(x.astype(jnp.bfloat16),
                 ((0, 0), (0, 0), (1, 1), (1, wq - w - 1)))
    xp = xp.reshape(n, c, (h + 2) * wq)
    xp = jnp.pad(xp, ((0, 0), (0, 0), (0, zl - (h + 2) * wq)))
    wt1 = w1.reshape(9 * c, c).T.astype(jnp.bfloat16)
    wt2 = w2.reshape(9 * c, c).T.astype(jnp.bfloat16)
    g1v = g1.astype(jnp.float32).reshape(c, 1)
    b1v = b1.astype(jnp.float32).reshape(c, 1)
    g2v = g2.astype(jnp.float32).reshape(c, 1)
    b2v = b2.astype(jnp.float32).reshape(c, 1)
    inv_m = 1.0 / (n * hw)

    zp_spec = pl.BlockSpec((gi, c, zl), lambda i: (i, 0, 0))
    pad_spec = pl.BlockSpec((gi, c, m), lambda i: (i, 0, 0))
    cmp_spec = pl.BlockSpec((gi, c, hw), lambda i: (i, 0, 0))
    stat_spec = pl.BlockSpec((1, c, 1), lambda i: (i, 0, 0))
    stat_full = pl.BlockSpec((steps, c, 1), lambda i: (0, 0, 0))
    w_spec = pl.BlockSpec((c, 9 * c), lambda i: (0, 0))
    vec_spec = pl.BlockSpec((c, 1), lambda i: (0, 0))
    params = pltpu.CompilerParams(dimension_semantics=("arbitrary",))
    f32 = jnp.float32
    bf16 = jnp.bfloat16

    y1, s1, ss1 = pl.pallas_call(
        functools.partial(_stage1, h=h, w=w, wq=wq, c=c, g_imgs=gi),
        grid=(steps,),
        in_specs=[zp_spec, w_spec],
        out_specs=(pad_spec, stat_spec, stat_spec),
        out_shape=(jax.ShapeDtypeStruct((n, c, m), bf16),
                   jax.ShapeDtypeStruct((steps, c, 1), f32),
                   jax.ShapeDtypeStruct((steps, c, 1), f32)),
        compiler_params=params,
    )(xp, wt1)

    y2, s2, ss2 = pl.pallas_call(
        functools.partial(_stage2, h=h, w=w, wq=wq, c=c, inv_m=inv_m,
                          g_imgs=gi),
        grid=(steps,),
        in_specs=[pad_spec, w_spec, stat_full, stat_full, vec_spec, vec_spec],
        out_specs=(pad_spec, stat_spec, stat_spec),
        out_shape=(jax.ShapeDtypeStruct((n, c, m), bf16),
                   jax.ShapeDtypeStruct((steps, c, 1), f32),
                   jax.ShapeDtypeStruct((steps, c, 1), f32)),
        scratch_shapes=[pltpu.VMEM((c, zl), jnp.bfloat16),
                        pltpu.VMEM((c, zl), jnp.bfloat16)],
        compiler_params=params,
    )(y1, wt2, s1, ss1, g1v, b1v)

    out = pl.pallas_call(
        functools.partial(_stage3, wq=wq, m=m, inv_m=inv_m, g_imgs=gi),
        grid=(steps,),
        in_specs=[pad_spec, stat_full, stat_full, vec_spec, vec_spec,
                  zp_spec],
        out_specs=pad_spec,
        out_shape=jax.ShapeDtypeStruct((n, c, m), f32),
        compiler_params=params,
    )(y2, s2, ss2, g2v, b2v, xp)

    return out.reshape(n, c, h, wq)[:, :, :, :w]


# speedup vs baseline: 1.2139x; 1.0508x over previous
import functools

import jax
import jax.numpy as jnp
from jax import lax
from jax.experimental import pallas as pl
from jax.experimental.pallas import tpu as pltpu

_EPS = 1e-5


def _next_pow2(v):
    p = 1
    while p < v:
        p *= 2
    return p


def _lane_mask(c, nlanes, wq, w):
    ln = lax.broadcasted_iota(jnp.int32, (c, nlanes), 1)
    return (ln & (wq - 1)) < w


def _conv_cm(z, wt_ref, lo, h, wq, c):
    m = h * wq
    base = lo - wq - 1
    acc = None
    for dy in range(3):
        for dx in range(3):
            s = base + dy * wq + dx
            t = dy * 3 + dx
            part = jnp.dot(wt_ref[:, t * c:(t + 1) * c], z[:, s:s + m],
                           preferred_element_type=jnp.float32)
            acc = part if acc is None else acc + part
    return acc


def _sum_wide(v, nlanes):
    if nlanes % 128 != 0:
        return jnp.sum(v, axis=1, keepdims=True)
    acc = v[:, 0:128]
    for i in range(1, nlanes // 128):
        acc = acc + v[:, i * 128:(i + 1) * 128]
    return acc


def _bn_coeffs(s_ref, ss_ref, g_ref, b_ref, inv_m):
    tot = jnp.sum(s_ref[...], axis=0)
    tot2 = jnp.sum(ss_ref[...], axis=0)
    mean = tot * inv_m
    var = jnp.maximum(tot2 * inv_m - mean * mean, 0.0)
    inv = lax.rsqrt(var + _EPS)
    scale = g_ref[...] * inv
    shift = b_ref[...] - mean * scale
    return scale, shift


def _stage1(xp_ref, wt_ref, y_ref, s_ref, ss_ref, *, h, w, wq, c, g_imgs):
    m = h * wq
    mask = _lane_mask(c, m, wq, w)
    ssum = None
    ssq = None
    for g in range(g_imgs):
        acc = _conv_cm(xp_ref[g], wt_ref, wq + 1, h, wq, c)
        accm = jnp.where(mask, acc, 0.0)
        y_ref[g] = accm.astype(y_ref.dtype)
        ps = _sum_wide(accm, m)
        pss = _sum_wide(accm * accm, m)
        ssum = ps if ssum is None else ssum + ps
        ssq = pss if ssq is None else ssq + pss
    s_ref[0] = jnp.sum(ssum, axis=1, keepdims=True)
    ss_ref[0] = jnp.sum(ssq, axis=1, keepdims=True)


def _stage2(y1_ref, wt_ref, s_ref, ss_ref, g_ref, b_ref,
            y2_ref, s2_ref, ss2_ref, za_ref, zb_ref,
            *, h, w, wq, c, inv_m, g_imgs):
    m = h * wq
    zl = za_ref.shape[1]
    lo = wq + 1
    mask = _lane_mask(c, m, wq, w)
    scale, shift = _bn_coeffs(s_ref, ss_ref, g_ref, b_ref, inv_m)
    zero_lo = jnp.zeros((c, lo), za_ref.dtype)
    zero_hi = jnp.zeros((c, zl - lo - m), za_ref.dtype)
    ssum = None
    ssq = None
    for g in range(g_imgs):
        z_ref = za_ref if g % 2 == 0 else zb_ref
        zb = jnp.where(mask, jnp.maximum(y1_ref[g] * scale + shift, 0.0), 0.0)
        z_ref[:, 0:lo] = zero_lo
        z_ref[:, lo:lo + m] = zb.astype(z_ref.dtype)
        z_ref[:, lo + m:] = zero_hi
        acc = _conv_cm(z_ref[...], wt_ref, lo, h, wq, c)
        accm = jnp.where(mask, acc, 0.0)
        y2_ref[g] = accm.astype(y2_ref.dtype)
        ps = _sum_wide(accm, m)
        pss = _sum_wide(accm * accm, m)
        ssum = ps if ssum is None else ssum + ps
        ssq = pss if ssq is None else ssq + pss
    s2_ref[0] = jnp.sum(ssum, axis=1, keepdims=True)
    ss2_ref[0] = jnp.sum(ssq, axis=1, keepdims=True)


def _stage3(y2_ref, s_ref, ss_ref, g_ref, b_ref, xp_ref, o_ref,
            *, wq, m, inv_m, g_imgs):
    lo = wq + 1
    scale, shift = _bn_coeffs(s_ref, ss_ref, g_ref, b_ref, inv_m)
    for g in range(g_imgs):
        res = xp_ref[g][:, lo:lo + m]
        y = y2_ref[g] * scale + shift + res
        o_ref[g] = jnp.maximum(y, 0.0).astype(o_ref.dtype)


def _pick_div(n, cap):
    for d in range(cap, 0, -1):
        if n % d == 0:
            return d
    return 1


def kernel(x, w1, cb1, g1, b1, w2, cb2, g2, b2):
    del cb1, cb2
    n, c, h, w = x.shape
    wq = _next_pow2(w + 2)
    m = h * wq
    hw = h * w
    zl = -(-(m + 2 * wq + 2) // 128) * 128
    gi = _pick_div(n, 8)
    steps = n // gi

    xp = jnp.pad(x.astype(jnp.bfloat16),
                 ((0, 0), (0, 0), (1, 1), (1, wq - w - 1)))
    xp = xp.reshape(n, c, (h + 2) * wq)
    xp = jnp.pad(xp, ((0, 0), (0, 0), (0, zl - (h + 2) * wq)))
    wt1 = w1.reshape(9 * c, c).T.astype(jnp.bfloat16)
    wt2 = w2.reshape(9 * c, c).T.astype(jnp.bfloat16)
    g1v = g1.astype(jnp.float32).reshape(c, 1)
    b1v = b1.astype(jnp.float32).reshape(c, 1)
    g2v = g2.astype(jnp.float32).reshape(c, 1)
    b2v = b2.astype(jnp.float32).reshape(c, 1)
    inv_m = 1.0 / (n * hw)

    zp_spec = pl.BlockSpec((gi, c, zl), lambda i: (i, 0, 0))
    pad_spec = pl.BlockSpec((gi, c, m), lambda i: (i, 0, 0))
    cmp_spec = pl.BlockSpec((gi, c, hw), lambda i: (i, 0, 0))
    stat_spec = pl.BlockSpec((1, c, 1), lambda i: (i, 0, 0))
    stat_full = pl.BlockSpec((steps, c, 1), lambda i: (0, 0, 0))
    w_spec = pl.BlockSpec((c, 9 * c), lambda i: (0, 0))
    vec_spec = pl.BlockSpec((c, 1), lambda i: (0, 0))
    params = pltpu.CompilerParams(dimension_semantics=("arbitrary",))
    f32 = jnp.float32
    bf16 = jnp.bfloat16

    y1, s1, ss1 = pl.pallas_call(
        functools.partial(_stage1, h=h, w=w, wq=wq, c=c, g_imgs=gi),
        grid=(steps,),
        in_specs=[zp_spec, w_spec],
        out_specs=(pad_spec, stat_spec, stat_spec),
        out_shape=(jax.ShapeDtypeStruct((n, c, m), bf16),
                   jax.ShapeDtypeStruct((steps, c, 1), f32),
                   jax.ShapeDtypeStruct((steps, c, 1), f32)),
        compiler_params=params,
    )(xp, wt1)

    y2, s2, ss2 = pl.pallas_call(
        functools.partial(_stage2, h=h, w=w, wq=wq, c=c, inv_m=inv_m,
                          g_imgs=gi),
        grid=(steps,),
        in_specs=[pad_spec, w_spec, stat_full, stat_full, vec_spec, vec_spec],
        out_specs=(pad_spec, stat_spec, stat_spec),
        out_shape=(jax.ShapeDtypeStruct((n, c, m), bf16),
                   jax.ShapeDtypeStruct((steps, c, 1), f32),
                   jax.ShapeDtypeStruct((steps, c, 1), f32)),
        scratch_shapes=[pltpu.VMEM((c, zl), jnp.bfloat16),
                        pltpu.VMEM((c, zl), jnp.bfloat16)],
        compiler_params=params,
    )(y1, wt2, s1, ss1, g1v, b1v)

    out = pl.pallas_call(
        functools.partial(_stage3, wq=wq, m=m, inv_m=inv_m, g_imgs=gi),
        grid=(steps,),
        in_specs=[pad_spec, stat_full, stat_full, vec_spec, vec_spec,
                  zp_spec],
        out_specs=pad_spec,
        out_shape=jax.ShapeDtypeStruct((n, c, m), f32),
        compiler_params=params,
    )(y2, s2, ss2, g2v, b2v, xp)

    return out.reshape(n, c, h, wq)[:, :, :, :w]


# speedup vs baseline: 1.2279x; 1.0115x over previous
import functools

import jax
import jax.numpy as jnp
from jax import lax
from jax.experimental import pallas as pl
from jax.experimental.pallas import tpu as pltpu

_EPS = 1e-5


def _next_pow2(v):
    p = 1
    while p < v:
        p *= 2
    return p


def _lane_mask(c, nlanes, wq, w):
    ln = lax.broadcasted_iota(jnp.int32, (c, nlanes), 1)
    return (ln & (wq - 1)) < w


def _conv_cm(z, wt_ref, lo, h, wq, c):
    m = h * wq
    base = lo - wq - 1
    acc = None
    for dy in range(3):
        for dx in range(3):
            s = base + dy * wq + dx
            t = dy * 3 + dx
            part = jnp.dot(wt_ref[:, t * c:(t + 1) * c], z[:, s:s + m],
                           preferred_element_type=jnp.float32)
            acc = part if acc is None else acc + part
    return acc


def _sum_wide(v, nlanes):
    if nlanes % 128 != 0:
        return jnp.sum(v, axis=1, keepdims=True)
    acc = v[:, 0:128]
    for i in range(1, nlanes // 128):
        acc = acc + v[:, i * 128:(i + 1) * 128]
    return acc


def _bn_coeffs(s_ref, ss_ref, g_ref, b_ref, inv_m):
    tot = jnp.sum(s_ref[...], axis=0)
    tot2 = jnp.sum(ss_ref[...], axis=0)
    mean = tot * inv_m
    var = jnp.maximum(tot2 * inv_m - mean * mean, 0.0)
    inv = lax.rsqrt(var + _EPS)
    scale = g_ref[...] * inv
    shift = b_ref[...] - mean * scale
    return scale, shift


def _stage1(xp_ref, wt_ref, y_ref, s_ref, ss_ref, *, h, w, wq, c, g_imgs):
    m = h * wq
    mask = _lane_mask(c, m, wq, w)
    ssum = None
    ssq = None
    for g in range(g_imgs):
        acc = _conv_cm(xp_ref[g], wt_ref, wq + 1, h, wq, c)
        accm = jnp.where(mask, acc, 0.0)
        y_ref[g] = accm.astype(y_ref.dtype)
        ps = _sum_wide(accm, m)
        pss = _sum_wide(accm * accm, m)
        ssum = ps if ssum is None else ssum + ps
        ssq = pss if ssq is None else ssq + pss
    s_ref[0] = jnp.sum(ssum, axis=1, keepdims=True)
    ss_ref[0] = jnp.sum(ssq, axis=1, keepdims=True)


def _stage2(y1_ref, wt_ref, s_ref, ss_ref, g_ref, b_ref,
            y2_ref, s2_ref, ss2_ref, za_ref, zb_ref,
            *, h, w, wq, c, inv_m, g_imgs):
    m = h * wq
    zl = za_ref.shape[1]
    lo = wq + 1
    mask = _lane_mask(c, m, wq, w)
    scale, shift = _bn_coeffs(s_ref, ss_ref, g_ref, b_ref, inv_m)
    zero_lo = jnp.zeros((c, lo), za_ref.dtype)
    zero_hi = jnp.zeros((c, zl - lo - m), za_ref.dtype)
    ssum = None
    ssq = None
    for g in range(g_imgs):
        z_ref = za_ref if g % 2 == 0 else zb_ref
        zb = jnp.where(mask, jnp.maximum(y1_ref[g] * scale + shift, 0.0), 0.0)
        z_ref[:, 0:lo] = zero_lo
        z_ref[:, lo:lo + m] = zb.astype(z_ref.dtype)
        z_ref[:, lo + m:] = zero_hi
        acc = _conv_cm(z_ref[...], wt_ref, lo, h, wq, c)
        accm = jnp.where(mask, acc, 0.0)
        y2_ref[g] = accm.astype(y2_ref.dtype)
        ps = _sum_wide(accm, m)
        pss = _sum_wide(accm * accm, m)
        ssum = ps if ssum is None else ssum + ps
        ssq = pss if ssq is None else ssq + pss
    s2_ref[0] = jnp.sum(ssum, axis=1, keepdims=True)
    ss2_ref[0] = jnp.sum(ssq, axis=1, keepdims=True)


def _stage3(y2_ref, s_ref, ss_ref, g_ref, b_ref, xp_ref, o_ref,
            *, wq, m, inv_m, g_imgs):
    lo = wq + 1
    scale, shift = _bn_coeffs(s_ref, ss_ref, g_ref, b_ref, inv_m)
    for g in range(g_imgs):
        res = xp_ref[g][:, lo:lo + m]
        y = y2_ref[g] * scale + shift + res
        o_ref[g] = jnp.maximum(y, 0.0).astype(o_ref.dtype)


def _pick_div(n, cap):
    for d in range(cap, 0, -1):
        if n % d == 0:
            return d
    return 1


def kernel(x, w1, cb1, g1, b1, w2, cb2, g2, b2):
    del cb1, cb2
    n, c, h, w = x.shape
    wq = _next_pow2(w + 2)
    m = h * wq
    hw = h * w
    zl = -(-(m + 2 * wq + 2) // 128) * 128
    gi = _pick_div(n, 16)
    steps = n // gi

    xp = jnp.pad(x.astype(jnp.bfloat16),
                 ((0, 0), (0, 0), (1, 1), (1, wq - w - 1)))
    xp = xp.reshape(n, c, (h + 2) * wq)
    xp = jnp.pad(xp, ((0, 0), (0, 0), (0, zl - (h + 2) * wq)))
    wt1 = w1.reshape(9 * c, c).T.astype(jnp.bfloat16)
    wt2 = w2.reshape(9 * c, c).T.astype(jnp.bfloat16)
    g1v = g1.astype(jnp.float32).reshape(c, 1)
    b1v = b1.astype(jnp.float32).reshape(c, 1)
    g2v = g2.astype(jnp.float32).reshape(c, 1)
    b2v = b2.astype(jnp.float32).reshape(c, 1)
    inv_m = 1.0 / (n * hw)

    zp_spec = pl.BlockSpec((gi, c, zl), lambda i: (i, 0, 0))
    pad_spec = pl.BlockSpec((gi, c, m), lambda i: (i, 0, 0))
    cmp_spec = pl.BlockSpec((gi, c, hw), lambda i: (i, 0, 0))
    stat_spec = pl.BlockSpec((1, c, 1), lambda i: (i, 0, 0))
    stat_full = pl.BlockSpec((steps, c, 1), lambda i: (0, 0, 0))
    w_spec = pl.BlockSpec((c, 9 * c), lambda i: (0, 0))
    vec_spec = pl.BlockSpec((c, 1), lambda i: (0, 0))
    params = pltpu.CompilerParams(dimension_semantics=("arbitrary",))
    f32 = jnp.float32
    bf16 = jnp.bfloat16

    y1, s1, ss1 = pl.pallas_call(
        functools.partial(_stage1, h=h, w=w, wq=wq, c=c, g_imgs=gi),
        grid=(steps,),
        in_specs=[zp_spec, w_spec],
        out_specs=(pad_spec, stat_spec, stat_spec),
        out_shape=(jax.ShapeDtypeStruct((n, c, m), bf16),
                   jax.ShapeDtypeStruct((steps, c, 1), f32),
                   jax.ShapeDtypeStruct((steps, c, 1), f32)),
        compiler_params=params,
    )(xp, wt1)

    y2, s2, ss2 = pl.pallas_call(
        functools.partial(_stage2, h=h, w=w, wq=wq, c=c, inv_m=inv_m,
                          g_imgs=gi),
        grid=(steps,),
        in_specs=[pad_spec, w_spec, stat_full, stat_full, vec_spec, vec_spec],
        out_specs=(pad_spec, stat_spec, stat_spec),
        out_shape=(jax.ShapeDtypeStruct((n, c, m), bf16),
                   jax.ShapeDtypeStruct((steps, c, 1), f32),
                   jax.ShapeDtypeStruct((steps, c, 1), f32)),
        scratch_shapes=[pltpu.VMEM((c, zl), jnp.bfloat16),
                        pltpu.VMEM((c, zl), jnp.bfloat16)],
        compiler_params=params,
    )(y1, wt2, s1, ss1, g1v, b1v)

    out = pl.pallas_call(
        functools.partial(_stage3, wq=wq, m=m, inv_m=inv_m, g_imgs=gi),
        grid=(steps,),
        in_specs=[pad_spec, stat_full, stat_full, vec_spec, vec_spec,
                  zp_spec],
        out_specs=pad_spec,
        out_shape=jax.ShapeDtypeStruct((n, c, m), f32),
        compiler_params=params,
    )(y2, s2, ss2, g2v, b2v, xp)

    return out.reshape(n, c, h, wq)[:, :, :, :w]
